# Initial kernel scaffold; baseline (speedup 1.0000x reference)
import jax
import jax.numpy as jnp
from jax.experimental import pallas as pl


def kernel(vertices, w0, d0, w1, b1, d1, w2, b2, d2, w3, b3, d3, w4, b4, d4, w5, b5, d5, w6, b6, d6):
    raise NotImplementedError("write your pallas kernel here")


# speedup vs baseline: 4.8949x; 4.8949x over previous
import functools
import math

import jax
import jax.numpy as jnp
from jax import lax
from jax.experimental import pallas as pl
from jax.experimental.pallas import tpu as pltpu
from jax.experimental.pallas import tpu_sc as plsc

_NC = 2
_NS = 16
_NW = _NC * _NS


def _knn_kernel(K, V, BR):
    def kf(v_ref, vt_ref, o_ref):
        b = pl.program_id(0)
        v = v_ref[0]
        vt = vt_ref[0]
        qr = jnp.sum(v * v, axis=1, keepdims=True)
        qc = jnp.sum(vt * vt, axis=0, keepdims=True)
        inner = (v[:, 0:1] * vt[0:1, :] + v[:, 1:2] * vt[1:2, :]
                 + v[:, 2:3] * vt[2:3, :])
        dist = -2.0 * inner + qc + qr
        iota = lax.broadcasted_iota(jnp.int32, (BR, V), 1)
        cols = []
        for _ in range(K):
            m = jnp.min(dist, axis=1, keepdims=True)
            sel = jnp.min(jnp.where(dist == m, iota, V), axis=1,
                          keepdims=True)
            cols.append(sel)
            dist = jnp.where(iota == sel, jnp.inf, dist)
        o_ref[0] = jnp.concatenate(cols, axis=1) + b * V
    return kf


def _knn(vertices, K):
    bs, V, _ = vertices.shape
    vt = jnp.swapaxes(vertices, 1, 2)
    BR = min(V, 256)
    nb = V // BR
    return pl.pallas_call(
        _knn_kernel(K, V, BR),
        grid=(bs, nb),
        in_specs=[
            pl.BlockSpec((1, BR, 3), lambda b, r: (b, r, 0)),
            pl.BlockSpec((1, 3, V), lambda b, r: (b, 0, 0)),
        ],
        out_specs=pl.BlockSpec((1, BR, K), lambda b, r: (b, r, 0)),
        out_shape=jax.ShapeDtypeStruct((bs, V, K), jnp.int32),
    )(vertices, vt)


def _sc_gather(table, idx):
    B, = idx.shape
    T, D = table.shape
    assert B % (8 * _NW) == 0
    bpw = B // _NW
    chunk = None
    for c in (128, 64, 32, 16, 8):
        if bpw % c == 0 and c * D * 4 <= 262144:
            chunk = c
            break
    n_iter = bpw // chunk
    mesh = plsc.VectorSubcoreMesh(core_axis_name="c", subcore_axis_name="s",
                                  num_cores=_NC, num_subcores=_NS)

    @functools.partial(
        pl.kernel,
        out_type=jax.ShapeDtypeStruct((B, D), jnp.float32),
        mesh=mesh,
        scratch_types=[
            pltpu.VMEM((chunk,), jnp.int32),
            pltpu.VMEM((chunk, D), jnp.float32),
            pltpu.SemaphoreType.DMA,
        ],
        compiler_params=pltpu.CompilerParams(use_tc_tiling_on_sc=False),
    )
    def k(table_hbm, idx_hbm, out_hbm, idx_v, rows_v, sem):
        wid = lax.axis_index("s") * _NC + lax.axis_index("c")
        base = wid * bpw

        def body(i, carry):
            off = base + i * chunk
            pltpu.sync_copy(idx_hbm.at[pl.ds(off, chunk)], idx_v)
            pltpu.async_copy(table_hbm.at[idx_v], rows_v, sem).wait()
            pltpu.sync_copy(rows_v, out_hbm.at[pl.ds(off, chunk)])
            return carry

        lax.fori_loop(0, n_iter, body, 0)

    return k(table, idx)


def _linear(x, w, b):
    R, Cin = x.shape
    Co = w.shape[1]
    BR = min(R, 2048)
    nb = R // BR

    def kf(x_ref, w_ref, b_ref, o_ref):
        o_ref[...] = lax.dot_general(
            x_ref[...], w_ref[...], (((1,), (0,)), ((), ())),
            preferred_element_type=jnp.float32) + b_ref[...]

    return pl.pallas_call(
        kf,
        grid=(nb,),
        in_specs=[
            pl.BlockSpec((BR, Cin), lambda r: (r, 0)),
            pl.BlockSpec((Cin, Co), lambda r: (0, 0)),
            pl.BlockSpec((1, Co), lambda r: (0, 0)),
        ],
        out_specs=pl.BlockSpec((BR, Co), lambda r: (r, 0)),
        out_shape=jax.ShapeDtypeStruct((R, Co), jnp.float32),
    )(x, w, b.reshape(1, Co))


def _agg(x, gv, dmat, gf=None, center=None, w0=None, relu_out=False,
         reduce_rows=False):
    R = x.shape[0]
    N = gv.shape[1]
    C = dmat.shape[1]
    if reduce_rows:
        BR = R
        nb = 1
    else:
        BR = None
        for c in (128, 64, 32, 16, 8, 4):
            if R % c == 0 and c * 24 * (128 + max(C, 128)) * 4 <= 6 * 2**20:
                BR = c
                break
        nb = R // BR

    def kf(*refs):
        it = iter(refs)
        x_ref = next(it)
        gv_ref = next(it)
        d_ref = next(it)
        gf_ref = next(it) if gf is not None else None
        c_ref = next(it) if center is not None else None
        w0_ref = next(it) if w0 is not None else None
        o_ref = next(it)
        d0 = d_ref[0:1, :]
        d1 = d_ref[1:2, :]
        d2 = d_ref[2:3, :]
        x0 = x_ref[:, 0:1]
        x1 = x_ref[:, 1:2]
        x2 = x_ref[:, 2:3]
        acc = None
        for n in range(N):
            g = gv_ref[:, n, :]
            dx = g[:, 0:1] - x0
            dy = g[:, 1:2] - x1
            dz = g[:, 2:3] - x2
            th = jnp.maximum(dx * d0 + dy * d1 + dz * d2, 0.0)
            if gf_ref is not None:
                th = th * gf_ref[:, n, :]
            acc = th if acc is None else jnp.maximum(acc, th)
        if w0_ref is not None:
            acc = acc * w0_ref[...]
        if c_ref is not None:
            acc = c_ref[...] + acc
        if relu_out:
            acc = jnp.maximum(acc, 0.0)
        if reduce_rows:
            rows = [jnp.max(acc[4 * g:4 * g + 4, :], axis=0, keepdims=True)
                    for g in range(R // 4)]
            o_ref[...] = jnp.concatenate(rows, axis=0)
        else:
            o_ref[...] = acc

    in_specs = [
        pl.BlockSpec((BR, 3), lambda r: (r, 0)),
        pl.BlockSpec((BR, N, 16), lambda r: (r, 0, 0)),
        pl.BlockSpec((3, C), lambda r: (0, 0)),
    ]
    args = [x, gv, dmat]
    if gf is not None:
        in_specs.append(pl.BlockSpec((BR, N, C), lambda r: (r, 0, 0)))
        args.append(gf)
    if center is not None:
        in_specs.append(pl.BlockSpec((BR, C), lambda r: (r, 0)))
        args.append(center)
    if w0 is not None:
        in_specs.append(pl.BlockSpec((1, C), lambda r: (0, 0)))
        args.append(w0)
    if reduce_rows:
        out_spec = pl.BlockSpec((R // 4, C), lambda r: (0, 0))
        out_shape = jax.ShapeDtypeStruct((R // 4, C), jnp.float32)
    else:
        out_spec = pl.BlockSpec((BR, C), lambda r: (r, 0))
        out_shape = jax.ShapeDtypeStruct((R, C), jnp.float32)
    return pl.pallas_call(
        kf,
        grid=(nb,),
        in_specs=in_specs,
        out_specs=out_spec,
        out_shape=out_shape,
    )(*args)


def _maxpool(gp):
    R, N, C = gp.shape
    BR = None
    for c in (256, 128, 64, 32, 16, 8, 4):
        if R % c == 0 and c * N * max(C, 128) * 4 <= 6 * 2**20:
            BR = c
            break
    nb = R // BR

    def kf(g_ref, o_ref):
        acc = g_ref[:, 0, :]
        for n in range(1, N):
            acc = jnp.maximum(acc, g_ref[:, n, :])
        o_ref[...] = acc

    return pl.pallas_call(
        kf,
        grid=(nb,),
        in_specs=[pl.BlockSpec((BR, N, C), lambda r: (r, 0, 0))],
        out_specs=pl.BlockSpec((BR, C), lambda r: (r, 0)),
        out_shape=jax.ShapeDtypeStruct((R, C), jnp.float32),
    )(gp)


def _pad16(v):
    return jnp.pad(v, ((0, 0), (0, 13)))


def _gather_padded(table, idx_flat):
    B = idx_flat.shape[0]
    Bp = ((B + 255) // 256) * 256
    if Bp != B:
        idx_flat = jnp.concatenate(
            [idx_flat, jnp.zeros((Bp - B,), jnp.int32)])
    out = _sc_gather(table, idx_flat)
    return out[:B]


def kernel(vertices, w0, d0, w1, b1, d1, w2, b2, d2, w3, b3, d3,
           w4, b4, d4, w5, b5, d5, w6, b6, d6):
    bs, V, _ = vertices.shape
    NB = 20

    ni1 = _knn(vertices, NB + 1)[:, :, 1:]
    ni1f = ni1.reshape(-1)
    vflat = vertices.reshape(-1, 3)
    gv1 = _sc_gather(_pad16(vflat), ni1f).reshape(bs * V, NB, 16)

    fm0 = _agg(vflat, gv1, d0, w0=w0.reshape(1, -1), relu_out=True)

    fo1 = _linear(fm0, w1, b1)
    gf1 = _sc_gather(fo1[:, 64:], ni1f).reshape(bs * V, NB, 64)
    fm1 = _agg(vflat, gv1, d1, gf=gf1, center=fo1[:, :64], relu_out=True)

    samp1 = jax.random.permutation(jax.random.key(101), V)[:V // 8]
    nip1 = ni1[:, samp1, :8].reshape(-1)
    fm1p = _maxpool(_sc_gather(fm1, nip1).reshape(-1, 8, 64))
    verts1 = vertices[:, samp1, :]
    vertices_anchor = verts1
    V1 = V // 8

    ni2 = _knn(verts1, NB + 1)[:, :, 1:]
    ni2f = ni2.reshape(-1)
    v1flat = verts1.reshape(-1, 3)
    gv2 = _sc_gather(_pad16(v1flat), ni2f).reshape(bs * V1, NB, 16)

    fo2 = _linear(fm1p, w2, b2)
    gf2 = _sc_gather(fo2[:, 128:], ni2f).reshape(bs * V1, NB, 128)
    fm2 = _agg(v1flat, gv2, d2, gf=gf2, center=fo2[:, :128], relu_out=True)

    fo3 = _linear(fm2, w3, b3)
    gf3 = _sc_gather(fo3[:, 256:], ni2f).reshape(bs * V1, NB, 256)
    fm3 = _agg(v1flat, gv2, d3, gf=gf3, center=fo3[:, :256], relu_out=True)

    samp2 = jax.random.permutation(jax.random.key(202), V1)[:V1 // 8]
    nip2 = ni2[:, samp2, :8].reshape(-1)
    fm3p = _maxpool(_sc_gather(fm3, nip2).reshape(-1, 8, 256))
    verts2 = verts1[:, samp2, :]
    V2 = V1 // 8

    ni3 = _knn(verts2, NB + 1)[:, :, 1:]
    ni3f = ni3.reshape(-1)
    v2flat = verts2.reshape(-1, 3)
    gv3 = _sc_gather(_pad16(v2flat), ni3f).reshape(bs * V2, NB, 16)

    fo4 = _linear(fm3p, w4, b4)
    gf4 = _sc_gather(fo4[:, 512:], ni3f).reshape(bs * V2, NB, 512)
    fm4 = _agg(v2flat, gv3, d4, gf=gf4, center=fo4[:, :512], relu_out=True)

    fo5 = _linear(fm4, w5, b5)
    gf5 = _sc_gather(fo5[:, 512:], ni3f).reshape(bs * V2, NB, 512)
    fm5 = _agg(v2flat, gv3, d5, gf=gf5, center=fo5[:, :512], relu_out=True)

    samp3 = jax.random.permutation(jax.random.key(303), V2)[:V2 // 8]
    nip3 = ni3[:, samp3, :8].reshape(-1)
    fm5p = _maxpool(_sc_gather(fm5, nip3).reshape(-1, 8, 512))
    verts3 = verts2[:, samp3, :]
    V3 = V2 // 8

    ni4 = _knn(verts3, 4)[:, :, 1:]
    ni4f = ni4.reshape(-1)
    v3flat = verts3.reshape(-1, 3)
    gv4 = _gather_padded(_pad16(v3flat), ni4f).reshape(bs * V3, 3, 16)

    fo6 = _linear(fm5p, w6, b6)
    gf6 = _gather_padded(fo6[:, 1024:], ni4f).reshape(bs * V3, 3, 1024)
    fg = _agg(v3flat, gv4, d6, gf=gf6, center=fo6[:, :1024],
              reduce_rows=True)
    return fg.reshape(bs, 1, 1024), vertices_anchor


# speedup vs baseline: 9.7257x; 1.9869x over previous
import functools
import math

import jax
import jax.numpy as jnp
from jax import lax
from jax.experimental import pallas as pl
from jax.experimental.pallas import tpu as pltpu
from jax.experimental.pallas import tpu_sc as plsc

_NC = 2
_NS = 16
_NW = _NC * _NS


def _knn_kernel(K, V, BR):
    def kf(v_ref, vt_ref, o_ref):
        b = pl.program_id(0)
        v = v_ref[0]
        vt = vt_ref[0]
        qr = jnp.sum(v * v, axis=1, keepdims=True)
        qc = jnp.sum(vt * vt, axis=0, keepdims=True)
        inner = (v[:, 0:1] * vt[0:1, :] + v[:, 1:2] * vt[1:2, :]
                 + v[:, 2:3] * vt[2:3, :])
        dist = -2.0 * inner + qc + qr
        iota = lax.broadcasted_iota(jnp.int32, (BR, V), 1)
        cols = []
        for _ in range(K):
            m = jnp.min(dist, axis=1, keepdims=True)
            sel = jnp.min(jnp.where(dist == m, iota, V), axis=1,
                          keepdims=True)
            cols.append(sel)
            dist = jnp.where(iota == sel, jnp.inf, dist)
        o_ref[0] = jnp.concatenate(cols, axis=1) + b * V
    return kf


def _knn(vertices, K):
    bs, V, _ = vertices.shape
    vt = jnp.swapaxes(vertices, 1, 2)
    BR = min(V, 256)
    nb = V // BR
    return pl.pallas_call(
        _knn_kernel(K, V, BR),
        grid=(bs, nb),
        in_specs=[
            pl.BlockSpec((1, BR, 3), lambda b, r: (b, r, 0)),
            pl.BlockSpec((1, 3, V), lambda b, r: (b, 0, 0)),
        ],
        out_specs=pl.BlockSpec((1, BR, K), lambda b, r: (b, r, 0)),
        out_shape=jax.ShapeDtypeStruct((bs, V, K), jnp.int32),
    )(vertices, vt)


def _sc_gather(table, idx):
    B, = idx.shape
    T, D = table.shape
    assert B % (8 * _NW) == 0
    bpw = B // _NW
    chunk = None
    for c in (128, 64, 32, 16, 8):
        if bpw % c == 0 and c * D * 4 <= 262144:
            chunk = c
            break
    n_iter = bpw // chunk
    mesh = plsc.VectorSubcoreMesh(core_axis_name="c", subcore_axis_name="s",
                                  num_cores=_NC, num_subcores=_NS)

    @functools.partial(
        pl.kernel,
        out_type=jax.ShapeDtypeStruct((B, D), jnp.float32),
        mesh=mesh,
        scratch_types=[
            pltpu.VMEM((chunk,), jnp.int32),
            pltpu.VMEM((chunk, D), jnp.float32),
            pltpu.SemaphoreType.DMA,
        ],
        compiler_params=pltpu.CompilerParams(use_tc_tiling_on_sc=False),
    )
    def k(table_hbm, idx_hbm, out_hbm, idx_v, rows_v, sem):
        wid = lax.axis_index("s") * _NC + lax.axis_index("c")
        base = wid * bpw

        def body(i, carry):
            off = base + i * chunk
            pltpu.sync_copy(idx_hbm.at[pl.ds(off, chunk)], idx_v)
            pltpu.async_copy(table_hbm.at[idx_v], rows_v, sem).wait()
            pltpu.sync_copy(rows_v, out_hbm.at[pl.ds(off, chunk)])
            return carry

        lax.fori_loop(0, n_iter, body, 0)

    return k(table, idx)


def _linear(x, w, b):
    R, Cin = x.shape
    Co = w.shape[1]
    BR = min(R, 2048)
    nb = R // BR

    def kf(x_ref, w_ref, b_ref, o_ref):
        o_ref[...] = lax.dot_general(
            x_ref[...], w_ref[...], (((1,), (0,)), ((), ())),
            preferred_element_type=jnp.float32) + b_ref[...]

    return pl.pallas_call(
        kf,
        grid=(nb,),
        in_specs=[
            pl.BlockSpec((BR, Cin), lambda r: (r, 0)),
            pl.BlockSpec((Cin, Co), lambda r: (0, 0)),
            pl.BlockSpec((1, Co), lambda r: (0, 0)),
        ],
        out_specs=pl.BlockSpec((BR, Co), lambda r: (r, 0)),
        out_shape=jax.ShapeDtypeStruct((R, Co), jnp.float32),
    )(x, w, b.reshape(1, Co))


def _agg(x, gv, dmat, gf=None, center=None, w0=None, relu_out=False,
         reduce_rows=False):
    R = x.shape[0]
    N = gv.shape[1]
    C = dmat.shape[1]
    if reduce_rows:
        BR = R
        nb = 1
    else:
        BR = None
        for c in (128, 64, 32, 16, 8, 4):
            if R % c == 0 and c * 24 * (128 + max(C, 128)) * 4 <= 6 * 2**20:
                BR = c
                break
        nb = R // BR

    def kf(*refs):
        it = iter(refs)
        x_ref = next(it)
        gv_ref = next(it)
        d_ref = next(it)
        gf_ref = next(it) if gf is not None else None
        c_ref = next(it) if center is not None else None
        w0_ref = next(it) if w0 is not None else None
        o_ref = next(it)
        dm = d_ref[...]
        xv = x_ref[...]
        acc = None
        for n in range(N):
            disp = gv_ref[:, n, :] - xv
            th = jnp.maximum(
                lax.dot_general(disp, dm, (((1,), (0,)), ((), ())),
                                preferred_element_type=jnp.float32), 0.0)
            if gf_ref is not None:
                th = th * gf_ref[:, n, :]
            acc = th if acc is None else jnp.maximum(acc, th)
        if w0_ref is not None:
            acc = acc * w0_ref[...]
        if c_ref is not None:
            acc = c_ref[...] + acc
        if relu_out:
            acc = jnp.maximum(acc, 0.0)
        if reduce_rows:
            rows = [jnp.max(acc[4 * g:4 * g + 4, :], axis=0, keepdims=True)
                    for g in range(R // 4)]
            o_ref[...] = jnp.concatenate(rows, axis=0)
        else:
            o_ref[...] = acc

    in_specs = [
        pl.BlockSpec((BR, 16), lambda r: (r, 0)),
        pl.BlockSpec((BR, N, 16), lambda r: (r, 0, 0)),
        pl.BlockSpec((16, C), lambda r: (0, 0)),
    ]
    args = [x, gv, jnp.pad(dmat, ((0, 13), (0, 0)))]
    if gf is not None:
        in_specs.append(pl.BlockSpec((BR, N, C), lambda r: (r, 0, 0)))
        args.append(gf)
    if center is not None:
        in_specs.append(pl.BlockSpec((BR, C), lambda r: (r, 0)))
        args.append(center)
    if w0 is not None:
        in_specs.append(pl.BlockSpec((1, C), lambda r: (0, 0)))
        args.append(w0)
    if reduce_rows:
        out_spec = pl.BlockSpec((R // 4, C), lambda r: (0, 0))
        out_shape = jax.ShapeDtypeStruct((R // 4, C), jnp.float32)
    else:
        out_spec = pl.BlockSpec((BR, C), lambda r: (r, 0))
        out_shape = jax.ShapeDtypeStruct((R, C), jnp.float32)
    return pl.pallas_call(
        kf,
        grid=(nb,),
        in_specs=in_specs,
        out_specs=out_spec,
        out_shape=out_shape,
    )(*args)


def _maxpool(gp):
    R, N, C = gp.shape
    BR = None
    for c in (256, 128, 64, 32, 16, 8, 4):
        if R % c == 0 and c * N * max(C, 128) * 4 <= 6 * 2**20:
            BR = c
            break
    nb = R // BR

    def kf(g_ref, o_ref):
        acc = g_ref[:, 0, :]
        for n in range(1, N):
            acc = jnp.maximum(acc, g_ref[:, n, :])
        o_ref[...] = acc

    return pl.pallas_call(
        kf,
        grid=(nb,),
        in_specs=[pl.BlockSpec((BR, N, C), lambda r: (r, 0, 0))],
        out_specs=pl.BlockSpec((BR, C), lambda r: (r, 0)),
        out_shape=jax.ShapeDtypeStruct((R, C), jnp.float32),
    )(gp)


def _pad16(v):
    return jnp.pad(v, ((0, 0), (0, 13)))


def _gather_padded(table, idx_flat):
    B = idx_flat.shape[0]
    Bp = ((B + 255) // 256) * 256
    if Bp != B:
        idx_flat = jnp.concatenate(
            [idx_flat, jnp.zeros((Bp - B,), jnp.int32)])
    out = _sc_gather(table, idx_flat)
    return out[:B]


def kernel(vertices, w0, d0, w1, b1, d1, w2, b2, d2, w3, b3, d3,
           w4, b4, d4, w5, b5, d5, w6, b6, d6):
    bs, V, _ = vertices.shape
    NB = 20

    ni1 = _knn(vertices, NB + 1)[:, :, 1:]
    ni1f = ni1.reshape(-1)
    vpad = _pad16(vertices.reshape(-1, 3))
    gv1 = _sc_gather(vpad, ni1f).reshape(bs * V, NB, 16)

    fm0 = _agg(vpad, gv1, d0, w0=w0.reshape(1, -1), relu_out=True)

    fo1 = _linear(fm0, w1, b1)
    gf1 = _sc_gather(fo1[:, 64:], ni1f).reshape(bs * V, NB, 64)
    fm1 = _agg(vpad, gv1, d1, gf=gf1, center=fo1[:, :64], relu_out=True)

    samp1 = jax.random.permutation(jax.random.key(101), V)[:V // 8]
    nip1 = ni1[:, samp1, :8].reshape(-1)
    fm1p = _maxpool(_sc_gather(fm1, nip1).reshape(-1, 8, 64))
    verts1 = vertices[:, samp1, :]
    vertices_anchor = verts1
    V1 = V // 8

    ni2 = _knn(verts1, NB + 1)[:, :, 1:]
    ni2f = ni2.reshape(-1)
    v1pad = _pad16(verts1.reshape(-1, 3))
    gv2 = _sc_gather(v1pad, ni2f).reshape(bs * V1, NB, 16)

    fo2 = _linear(fm1p, w2, b2)
    gf2 = _sc_gather(fo2[:, 128:], ni2f).reshape(bs * V1, NB, 128)
    fm2 = _agg(v1pad, gv2, d2, gf=gf2, center=fo2[:, :128], relu_out=True)

    fo3 = _linear(fm2, w3, b3)
    gf3 = _sc_gather(fo3[:, 256:], ni2f).reshape(bs * V1, NB, 256)
    fm3 = _agg(v1pad, gv2, d3, gf=gf3, center=fo3[:, :256], relu_out=True)

    samp2 = jax.random.permutation(jax.random.key(202), V1)[:V1 // 8]
    nip2 = ni2[:, samp2, :8].reshape(-1)
    fm3p = _maxpool(_sc_gather(fm3, nip2).reshape(-1, 8, 256))
    verts2 = verts1[:, samp2, :]
    V2 = V1 // 8

    ni3 = _knn(verts2, NB + 1)[:, :, 1:]
    ni3f = ni3.reshape(-1)
    v2pad = _pad16(verts2.reshape(-1, 3))
    gv3 = _sc_gather(v2pad, ni3f).reshape(bs * V2, NB, 16)

    fo4 = _linear(fm3p, w4, b4)
    gf4 = _sc_gather(fo4[:, 512:], ni3f).reshape(bs * V2, NB, 512)
    fm4 = _agg(v2pad, gv3, d4, gf=gf4, center=fo4[:, :512], relu_out=True)

    fo5 = _linear(fm4, w5, b5)
    gf5 = _sc_gather(fo5[:, 512:], ni3f).reshape(bs * V2, NB, 512)
    fm5 = _agg(v2pad, gv3, d5, gf=gf5, center=fo5[:, :512], relu_out=True)

    samp3 = jax.random.permutation(jax.random.key(303), V2)[:V2 // 8]
    nip3 = ni3[:, samp3, :8].reshape(-1)
    fm5p = _maxpool(_sc_gather(fm5, nip3).reshape(-1, 8, 512))
    verts3 = verts2[:, samp3, :]
    V3 = V2 // 8

    ni4 = _knn(verts3, 4)[:, :, 1:]
    ni4f = ni4.reshape(-1)
    v3pad = _pad16(verts3.reshape(-1, 3))
    gv4 = _gather_padded(v3pad, ni4f).reshape(bs * V3, 3, 16)

    fo6 = _linear(fm5p, w6, b6)
    gf6 = _gather_padded(fo6[:, 1024:], ni4f).reshape(bs * V3, 3, 1024)
    fg = _agg(v3pad, gv4, d6, gf=gf6, center=fo6[:, :1024],
              reduce_rows=True)
    return fg.reshape(bs, 1, 1024), vertices_anchor


# speedup vs baseline: 10.2041x; 1.0492x over previous
import functools
import math

import jax
import jax.numpy as jnp
from jax import lax
from jax.experimental import pallas as pl
from jax.experimental.pallas import tpu as pltpu
from jax.experimental.pallas import tpu_sc as plsc

_NC = 2
_NS = 16
_NW = _NC * _NS


def _knn_kernel(K, V, BR):
    def kf(v_ref, vt_ref, o_ref):
        b = pl.program_id(0)
        v = v_ref[0]
        vt = vt_ref[0]
        qr = jnp.sum(v * v, axis=1, keepdims=True)
        qc = jnp.sum(vt * vt, axis=0, keepdims=True)
        inner = (v[:, 0:1] * vt[0:1, :] + v[:, 1:2] * vt[1:2, :]
                 + v[:, 2:3] * vt[2:3, :])
        dist = -2.0 * inner + qc + qr
        iota = lax.broadcasted_iota(jnp.int32, (BR, V), 1)
        cols = []
        for _ in range(K):
            m = jnp.min(dist, axis=1, keepdims=True)
            sel = jnp.min(jnp.where(dist == m, iota, V), axis=1,
                          keepdims=True)
            cols.append(sel)
            dist = jnp.where(iota == sel, jnp.inf, dist)
        o_ref[0] = jnp.concatenate(cols, axis=1) + b * V
    return kf


def _knn(vertices, K):
    bs, V, _ = vertices.shape
    vt = jnp.swapaxes(vertices, 1, 2)
    BR = min(V, 256)
    nb = V // BR
    return pl.pallas_call(
        _knn_kernel(K, V, BR),
        grid=(bs, nb),
        in_specs=[
            pl.BlockSpec((1, BR, 3), lambda b, r: (b, r, 0)),
            pl.BlockSpec((1, 3, V), lambda b, r: (b, 0, 0)),
        ],
        out_specs=pl.BlockSpec((1, BR, K), lambda b, r: (b, r, 0)),
        out_shape=jax.ShapeDtypeStruct((bs, V, K), jnp.int32),
    )(vertices, vt)


def _sc_gather(table, idx):
    B, = idx.shape
    T, D = table.shape
    assert B % (8 * _NW) == 0
    bpw = B // _NW
    chunk = None
    for c in (128, 64, 32, 16, 8):
        if bpw % c == 0 and c * D * 4 <= 262144 and (bpw == c
                                                     or (bpw // c) % 2 == 0):
            chunk = c
            break
    if chunk is None:
        chunk = 8
    n_iter = bpw // chunk
    pipelined = n_iter > 1 and n_iter % 2 == 0
    mesh = plsc.VectorSubcoreMesh(core_axis_name="c", subcore_axis_name="s",
                                  num_cores=_NC, num_subcores=_NS)

    @functools.partial(
        pl.kernel,
        out_type=jax.ShapeDtypeStruct((B, D), jnp.float32),
        mesh=mesh,
        scratch_types=[
            pltpu.VMEM((bpw,), jnp.int32),
            pltpu.VMEM((chunk, D), jnp.float32),
            pltpu.VMEM((chunk, D), jnp.float32),
            pltpu.SemaphoreType.DMA,
            pltpu.SemaphoreType.DMA,
        ],
        compiler_params=pltpu.CompilerParams(use_tc_tiling_on_sc=False),
    )
    def k(table_hbm, idx_hbm, out_hbm, idx_v, buf0, buf1, sem0, sem1):
        wid = lax.axis_index("s") * _NC + lax.axis_index("c")
        base = wid * bpw
        pltpu.sync_copy(idx_hbm.at[pl.ds(base, bpw)], idx_v)

        def gather(i, buf, sem):
            return pltpu.async_copy(
                table_hbm.at[idx_v.at[pl.ds(i * chunk, chunk)]], buf, sem)

        if not pipelined:
            if n_iter == 1:
                gather(0, buf0, sem0).wait()
                pltpu.sync_copy(buf0, out_hbm.at[pl.ds(base, chunk)])
            else:
                def body(i, carry):
                    gather(i, buf0, sem0).wait()
                    pltpu.sync_copy(
                        buf0, out_hbm.at[pl.ds(base + i * chunk, chunk)])
                    return carry
                lax.fori_loop(0, n_iter, body, 0)
        else:
            gather(0, buf0, sem0)

            def body(j, carry):
                i0 = 2 * j
                pltpu.make_async_copy(
                    table_hbm.at[idx_v.at[pl.ds(i0 * chunk, chunk)]],
                    buf0, sem0).wait()
                gather(i0 + 1, buf1, sem1)
                pltpu.sync_copy(
                    buf0, out_hbm.at[pl.ds(base + i0 * chunk, chunk)])
                pltpu.make_async_copy(
                    table_hbm.at[idx_v.at[pl.ds((i0 + 1) * chunk, chunk)]],
                    buf1, sem1).wait()

                @pl.when(i0 + 2 < n_iter)
                def _():
                    gather(i0 + 2, buf0, sem0)

                pltpu.sync_copy(
                    buf1, out_hbm.at[pl.ds(base + (i0 + 1) * chunk, chunk)])
                return carry

            lax.fori_loop(0, n_iter // 2, body, 0)

    return k(table, idx)


def _linear(x, w, b):
    R, Cin = x.shape
    Co = w.shape[1]
    BR = min(R, 2048)
    nb = R // BR

    def kf(x_ref, w_ref, b_ref, o_ref):
        o_ref[...] = lax.dot_general(
            x_ref[...], w_ref[...], (((1,), (0,)), ((), ())),
            preferred_element_type=jnp.float32) + b_ref[...]

    return pl.pallas_call(
        kf,
        grid=(nb,),
        in_specs=[
            pl.BlockSpec((BR, Cin), lambda r: (r, 0)),
            pl.BlockSpec((Cin, Co), lambda r: (0, 0)),
            pl.BlockSpec((1, Co), lambda r: (0, 0)),
        ],
        out_specs=pl.BlockSpec((BR, Co), lambda r: (r, 0)),
        out_shape=jax.ShapeDtypeStruct((R, Co), jnp.float32),
    )(x, w, b.reshape(1, Co))


def _agg(x, gv, dmat, gf=None, center=None, w0=None, relu_out=False,
         reduce_rows=False):
    R = x.shape[0]
    N = gv.shape[1]
    C = dmat.shape[1]
    if reduce_rows:
        BR = R
        nb = 1
    else:
        BR = None
        for c in (128, 64, 32, 16, 8, 4):
            if R % c == 0 and c * 24 * (128 + max(C, 128)) * 4 <= 6 * 2**20:
                BR = c
                break
        nb = R // BR

    def kf(*refs):
        it = iter(refs)
        x_ref = next(it)
        gv_ref = next(it)
        d_ref = next(it)
        gf_ref = next(it) if gf is not None else None
        c_ref = next(it) if center is not None else None
        w0_ref = next(it) if w0 is not None else None
        o_ref = next(it)
        dm = d_ref[...]
        xv = x_ref[...]
        acc = None
        for n in range(N):
            disp = gv_ref[:, n, :] - xv
            th = jnp.maximum(
                lax.dot_general(disp, dm, (((1,), (0,)), ((), ())),
                                preferred_element_type=jnp.float32), 0.0)
            if gf_ref is not None:
                th = th * gf_ref[:, n, :]
            acc = th if acc is None else jnp.maximum(acc, th)
        if w0_ref is not None:
            acc = acc * w0_ref[...]
        if c_ref is not None:
            acc = c_ref[...] + acc
        if relu_out:
            acc = jnp.maximum(acc, 0.0)
        if reduce_rows:
            rows = [jnp.max(acc[4 * g:4 * g + 4, :], axis=0, keepdims=True)
                    for g in range(R // 4)]
            o_ref[...] = jnp.concatenate(rows, axis=0)
        else:
            o_ref[...] = acc

    in_specs = [
        pl.BlockSpec((BR, 16), lambda r: (r, 0)),
        pl.BlockSpec((BR, N, 16), lambda r: (r, 0, 0)),
        pl.BlockSpec((16, C), lambda r: (0, 0)),
    ]
    args = [x, gv, jnp.pad(dmat, ((0, 13), (0, 0)))]
    if gf is not None:
        in_specs.append(pl.BlockSpec((BR, N, C), lambda r: (r, 0, 0)))
        args.append(gf)
    if center is not None:
        in_specs.append(pl.BlockSpec((BR, C), lambda r: (r, 0)))
        args.append(center)
    if w0 is not None:
        in_specs.append(pl.BlockSpec((1, C), lambda r: (0, 0)))
        args.append(w0)
    if reduce_rows:
        out_spec = pl.BlockSpec((R // 4, C), lambda r: (0, 0))
        out_shape = jax.ShapeDtypeStruct((R // 4, C), jnp.float32)
    else:
        out_spec = pl.BlockSpec((BR, C), lambda r: (r, 0))
        out_shape = jax.ShapeDtypeStruct((R, C), jnp.float32)
    return pl.pallas_call(
        kf,
        grid=(nb,),
        in_specs=in_specs,
        out_specs=out_spec,
        out_shape=out_shape,
    )(*args)


def _maxpool(gp):
    R, N, C = gp.shape
    BR = None
    for c in (256, 128, 64, 32, 16, 8, 4):
        if R % c == 0 and c * N * max(C, 128) * 4 <= 6 * 2**20:
            BR = c
            break
    nb = R // BR

    def kf(g_ref, o_ref):
        acc = g_ref[:, 0, :]
        for n in range(1, N):
            acc = jnp.maximum(acc, g_ref[:, n, :])
        o_ref[...] = acc

    return pl.pallas_call(
        kf,
        grid=(nb,),
        in_specs=[pl.BlockSpec((BR, N, C), lambda r: (r, 0, 0))],
        out_specs=pl.BlockSpec((BR, C), lambda r: (r, 0)),
        out_shape=jax.ShapeDtypeStruct((R, C), jnp.float32),
    )(gp)


def _pad16(v):
    return jnp.pad(v, ((0, 0), (0, 13)))


def _gather_padded(table, idx_flat):
    B = idx_flat.shape[0]
    Bp = ((B + 255) // 256) * 256
    if Bp != B:
        idx_flat = jnp.concatenate(
            [idx_flat, jnp.zeros((Bp - B,), jnp.int32)])
    out = _sc_gather(table, idx_flat)
    return out[:B]


def kernel(vertices, w0, d0, w1, b1, d1, w2, b2, d2, w3, b3, d3,
           w4, b4, d4, w5, b5, d5, w6, b6, d6):
    bs, V, _ = vertices.shape
    NB = 20

    ni1 = _knn(vertices, NB + 1)[:, :, 1:]
    ni1f = ni1.reshape(-1)
    vpad = _pad16(vertices.reshape(-1, 3))
    gv1 = _sc_gather(vpad, ni1f).reshape(bs * V, NB, 16)

    fm0 = _agg(vpad, gv1, d0, w0=w0.reshape(1, -1), relu_out=True)

    fo1 = _linear(fm0, w1, b1)
    gf1 = _sc_gather(fo1[:, 64:], ni1f).reshape(bs * V, NB, 64)
    fm1 = _agg(vpad, gv1, d1, gf=gf1, center=fo1[:, :64], relu_out=True)

    samp1 = jax.random.permutation(jax.random.key(101), V)[:V // 8]
    nip1 = ni1[:, samp1, :8].reshape(-1)
    fm1p = _maxpool(_sc_gather(fm1, nip1).reshape(-1, 8, 64))
    verts1 = vertices[:, samp1, :]
    vertices_anchor = verts1
    V1 = V // 8

    ni2 = _knn(verts1, NB + 1)[:, :, 1:]
    ni2f = ni2.reshape(-1)
    v1pad = _pad16(verts1.reshape(-1, 3))
    gv2 = _sc_gather(v1pad, ni2f).reshape(bs * V1, NB, 16)

    fo2 = _linear(fm1p, w2, b2)
    gf2 = _sc_gather(fo2[:, 128:], ni2f).reshape(bs * V1, NB, 128)
    fm2 = _agg(v1pad, gv2, d2, gf=gf2, center=fo2[:, :128], relu_out=True)

    fo3 = _linear(fm2, w3, b3)
    gf3 = _sc_gather(fo3[:, 256:], ni2f).reshape(bs * V1, NB, 256)
    fm3 = _agg(v1pad, gv2, d3, gf=gf3, center=fo3[:, :256], relu_out=True)

    samp2 = jax.random.permutation(jax.random.key(202), V1)[:V1 // 8]
    nip2 = ni2[:, samp2, :8].reshape(-1)
    fm3p = _maxpool(_sc_gather(fm3, nip2).reshape(-1, 8, 256))
    verts2 = verts1[:, samp2, :]
    V2 = V1 // 8

    ni3 = _knn(verts2, NB + 1)[:, :, 1:]
    ni3f = ni3.reshape(-1)
    v2pad = _pad16(verts2.reshape(-1, 3))
    gv3 = _sc_gather(v2pad, ni3f).reshape(bs * V2, NB, 16)

    fo4 = _linear(fm3p, w4, b4)
    gf4 = _sc_gather(fo4[:, 512:], ni3f).reshape(bs * V2, NB, 512)
    fm4 = _agg(v2pad, gv3, d4, gf=gf4, center=fo4[:, :512], relu_out=True)

    fo5 = _linear(fm4, w5, b5)
    gf5 = _sc_gather(fo5[:, 512:], ni3f).reshape(bs * V2, NB, 512)
    fm5 = _agg(v2pad, gv3, d5, gf=gf5, center=fo5[:, :512], relu_out=True)

    samp3 = jax.random.permutation(jax.random.key(303), V2)[:V2 // 8]
    nip3 = ni3[:, samp3, :8].reshape(-1)
    fm5p = _maxpool(_sc_gather(fm5, nip3).reshape(-1, 8, 512))
    verts3 = verts2[:, samp3, :]
    V3 = V2 // 8

    ni4 = _knn(verts3, 4)[:, :, 1:]
    ni4f = ni4.reshape(-1)
    v3pad = _pad16(verts3.reshape(-1, 3))
    gv4 = _gather_padded(v3pad, ni4f).reshape(bs * V3, 3, 16)

    fo6 = _linear(fm5p, w6, b6)
    gf6 = _gather_padded(fo6[:, 1024:], ni4f).reshape(bs * V3, 3, 1024)
    fg = _agg(v3pad, gv4, d6, gf=gf6, center=fo6[:, :1024],
              reduce_rows=True)
    return fg.reshape(bs, 1, 1024), vertices_anchor


# speedup vs baseline: 10.2975x; 1.0091x over previous
import functools
import math

import jax
import jax.numpy as jnp
from jax import lax
from jax.experimental import pallas as pl
from jax.experimental.pallas import tpu as pltpu
from jax.experimental.pallas import tpu_sc as plsc

_NC = 2
_NS = 16
_NW = _NC * _NS


def _knn_kernel(K, V, BR):
    def kf(v_ref, vt_ref, o_ref):
        b = pl.program_id(0)
        v = v_ref[0]
        vt = vt_ref[0]
        qr = jnp.sum(v * v, axis=1, keepdims=True)
        qc = jnp.sum(vt * vt, axis=0, keepdims=True)
        inner = (v[:, 0:1] * vt[0:1, :] + v[:, 1:2] * vt[1:2, :]
                 + v[:, 2:3] * vt[2:3, :])
        dist = -2.0 * inner + qc + qr
        iota = lax.broadcasted_iota(jnp.int32, (BR, V), 1)
        cols = []
        for _ in range(K):
            m = jnp.min(dist, axis=1, keepdims=True)
            sel = jnp.min(jnp.where(dist == m, iota, V), axis=1,
                          keepdims=True)
            cols.append(sel)
            dist = jnp.where(iota == sel, jnp.inf, dist)
        o_ref[0] = jnp.concatenate(cols, axis=1) + b * V
    return kf


def _knn(vertices, K):
    bs, V, _ = vertices.shape
    vt = jnp.swapaxes(vertices, 1, 2)
    BR = min(V, 256)
    nb = V // BR
    return pl.pallas_call(
        _knn_kernel(K, V, BR),
        grid=(bs, nb),
        in_specs=[
            pl.BlockSpec((1, BR, 3), lambda b, r: (b, r, 0)),
            pl.BlockSpec((1, 3, V), lambda b, r: (b, 0, 0)),
        ],
        out_specs=pl.BlockSpec((1, BR, K), lambda b, r: (b, r, 0)),
        out_shape=jax.ShapeDtypeStruct((bs, V, K), jnp.int32),
    )(vertices, vt)


def _sc_gather(table, idx):
    B, = idx.shape
    T, D = table.shape
    assert B % (8 * _NW) == 0
    bpw = B // _NW
    chunk = None
    for c in (128, 64, 32, 16, 8):
        if bpw % c == 0 and c * D * 4 <= 262144 and (bpw == c
                                                     or (bpw // c) % 2 == 0):
            chunk = c
            break
    if chunk is None:
        chunk = 8
    n_iter = bpw // chunk
    pipelined = n_iter > 1 and n_iter % 2 == 0
    mesh = plsc.VectorSubcoreMesh(core_axis_name="c", subcore_axis_name="s",
                                  num_cores=_NC, num_subcores=_NS)

    @functools.partial(
        pl.kernel,
        out_type=jax.ShapeDtypeStruct((B, D), jnp.float32),
        mesh=mesh,
        scratch_types=[
            pltpu.VMEM((bpw,), jnp.int32),
            pltpu.VMEM((chunk, D), jnp.float32),
            pltpu.VMEM((chunk, D), jnp.float32),
            pltpu.SemaphoreType.DMA,
            pltpu.SemaphoreType.DMA,
        ],
        compiler_params=pltpu.CompilerParams(use_tc_tiling_on_sc=False),
    )
    def k(table_hbm, idx_hbm, out_hbm, idx_v, buf0, buf1, sem0, sem1):
        wid = lax.axis_index("s") * _NC + lax.axis_index("c")
        base = wid * bpw
        pltpu.sync_copy(idx_hbm.at[pl.ds(base, bpw)], idx_v)

        def gather(i, buf, sem):
            return pltpu.async_copy(
                table_hbm.at[idx_v.at[pl.ds(i * chunk, chunk)]], buf, sem)

        if not pipelined:
            if n_iter == 1:
                gather(0, buf0, sem0).wait()
                pltpu.sync_copy(buf0, out_hbm.at[pl.ds(base, chunk)])
            else:
                def body(i, carry):
                    gather(i, buf0, sem0).wait()
                    pltpu.sync_copy(
                        buf0, out_hbm.at[pl.ds(base + i * chunk, chunk)])
                    return carry
                lax.fori_loop(0, n_iter, body, 0)
        else:
            gather(0, buf0, sem0)

            def body(j, carry):
                i0 = 2 * j
                pltpu.make_async_copy(
                    table_hbm.at[idx_v.at[pl.ds(i0 * chunk, chunk)]],
                    buf0, sem0).wait()
                gather(i0 + 1, buf1, sem1)
                pltpu.sync_copy(
                    buf0, out_hbm.at[pl.ds(base + i0 * chunk, chunk)])
                pltpu.make_async_copy(
                    table_hbm.at[idx_v.at[pl.ds((i0 + 1) * chunk, chunk)]],
                    buf1, sem1).wait()

                @pl.when(i0 + 2 < n_iter)
                def _():
                    gather(i0 + 2, buf0, sem0)

                pltpu.sync_copy(
                    buf1, out_hbm.at[pl.ds(base + (i0 + 1) * chunk, chunk)])
                return carry

            lax.fori_loop(0, n_iter // 2, body, 0)

    return k(table, idx)


def _linear_epilogue(fm, wl_ref, bl_ref, oc_ref, os_ref):
    fo = lax.dot_general(fm, wl_ref[...], (((1,), (0,)), ((), ())),
                         preferred_element_type=jnp.float32) + bl_ref[...]
    half = fo.shape[1] // 2
    oc_ref[...] = fo[:, :half]
    os_ref[...] = fo[:, half:]


def _agg(x, gv, dmat, gf=None, center=None, w0=None, relu_out=False,
         reduce_rows=False, wl=None, bl=None):
    R = x.shape[0]
    N = gv.shape[1]
    C = dmat.shape[1]
    if reduce_rows:
        BR = R
        nb = 1
    else:
        BR = None
        for c in (128, 64, 32, 16, 8, 4):
            if R % c == 0 and c * 24 * (128 + max(C, 128)) * 4 <= 6 * 2**20:
                BR = c
                break
        nb = R // BR

    def kf(*refs):
        it = iter(refs)
        x_ref = next(it)
        gv_ref = next(it)
        d_ref = next(it)
        gf_ref = next(it) if gf is not None else None
        c_ref = next(it) if center is not None else None
        w0_ref = next(it) if w0 is not None else None
        wl_ref = next(it) if wl is not None else None
        bl_ref = next(it) if wl is not None else None
        o_refs = list(it)
        dm = d_ref[...]
        xv = x_ref[...]
        acc = None
        for n in range(N):
            disp = gv_ref[:, n, :] - xv
            th = jnp.maximum(
                lax.dot_general(disp, dm, (((1,), (0,)), ((), ())),
                                preferred_element_type=jnp.float32), 0.0)
            if gf_ref is not None:
                th = th * gf_ref[:, n, :]
            acc = th if acc is None else jnp.maximum(acc, th)
        if w0_ref is not None:
            acc = acc * w0_ref[...]
        if c_ref is not None:
            acc = c_ref[...] + acc
        if relu_out:
            acc = jnp.maximum(acc, 0.0)
        if reduce_rows:
            rows = [jnp.max(acc[4 * g:4 * g + 4, :], axis=0, keepdims=True)
                    for g in range(R // 4)]
            o_refs[0][...] = jnp.concatenate(rows, axis=0)
        elif wl is not None:
            _linear_epilogue(acc, wl_ref, bl_ref, o_refs[0], o_refs[1])
        else:
            o_refs[0][...] = acc

    in_specs = [
        pl.BlockSpec((BR, 16), lambda r: (r, 0)),
        pl.BlockSpec((BR, N, 16), lambda r: (r, 0, 0)),
        pl.BlockSpec((16, C), lambda r: (0, 0)),
    ]
    args = [x, gv, jnp.pad(dmat, ((0, 13), (0, 0)))]
    if gf is not None:
        in_specs.append(pl.BlockSpec((BR, N, C), lambda r: (r, 0, 0)))
        args.append(gf)
    if center is not None:
        in_specs.append(pl.BlockSpec((BR, C), lambda r: (r, 0)))
        args.append(center)
    if w0 is not None:
        in_specs.append(pl.BlockSpec((1, C), lambda r: (0, 0)))
        args.append(w0)
    if wl is not None:
        Co = wl.shape[1] // 2
        in_specs.append(pl.BlockSpec(wl.shape, lambda r: (0, 0)))
        in_specs.append(pl.BlockSpec((1, 2 * Co), lambda r: (0, 0)))
        args += [wl, bl.reshape(1, -1)]
        out_spec = [pl.BlockSpec((BR, Co), lambda r: (r, 0))] * 2
        out_shape = [jax.ShapeDtypeStruct((R, Co), jnp.float32)] * 2
    elif reduce_rows:
        out_spec = pl.BlockSpec((R // 4, C), lambda r: (0, 0))
        out_shape = jax.ShapeDtypeStruct((R // 4, C), jnp.float32)
    else:
        out_spec = pl.BlockSpec((BR, C), lambda r: (r, 0))
        out_shape = jax.ShapeDtypeStruct((R, C), jnp.float32)
    return pl.pallas_call(
        kf,
        grid=(nb,),
        in_specs=in_specs,
        out_specs=out_spec,
        out_shape=out_shape,
    )(*args)


def _maxpool(gp, wl, bl):
    R, N, C = gp.shape
    BR = None
    for c in (256, 128, 64, 32, 16, 8, 4):
        if R % c == 0 and c * N * max(C, 128) * 4 <= 6 * 2**20:
            BR = c
            break
    nb = R // BR
    Co = wl.shape[1] // 2

    def kf(g_ref, wl_ref, bl_ref, oc_ref, os_ref):
        acc = g_ref[:, 0, :]
        for n in range(1, N):
            acc = jnp.maximum(acc, g_ref[:, n, :])
        _linear_epilogue(acc, wl_ref, bl_ref, oc_ref, os_ref)

    return pl.pallas_call(
        kf,
        grid=(nb,),
        in_specs=[
            pl.BlockSpec((BR, N, C), lambda r: (r, 0, 0)),
            pl.BlockSpec(wl.shape, lambda r: (0, 0)),
            pl.BlockSpec((1, 2 * Co), lambda r: (0, 0)),
        ],
        out_specs=[pl.BlockSpec((BR, Co), lambda r: (r, 0))] * 2,
        out_shape=[jax.ShapeDtypeStruct((R, Co), jnp.float32)] * 2,
    )(gp, wl, bl.reshape(1, -1))


def _pad16(v):
    return jnp.pad(v, ((0, 0), (0, 13)))


def _gather_padded(table, idx_flat):
    B = idx_flat.shape[0]
    Bp = ((B + 255) // 256) * 256
    if Bp != B:
        idx_flat = jnp.concatenate(
            [idx_flat, jnp.zeros((Bp - B,), jnp.int32)])
    out = _sc_gather(table, idx_flat)
    return out[:B]


def kernel(vertices, w0, d0, w1, b1, d1, w2, b2, d2, w3, b3, d3,
           w4, b4, d4, w5, b5, d5, w6, b6, d6):
    bs, V, _ = vertices.shape
    NB = 20

    ni1 = _knn(vertices, NB + 1)[:, :, 1:]
    ni1f = ni1.reshape(-1)
    vpad = _pad16(vertices.reshape(-1, 3))
    gv1 = _sc_gather(vpad, ni1f).reshape(bs * V, NB, 16)

    c1, s1 = _agg(vpad, gv1, d0, w0=w0.reshape(1, -1), relu_out=True,
                  wl=w1, bl=b1)
    gf1 = _sc_gather(s1, ni1f).reshape(bs * V, NB, 64)
    fm1 = _agg(vpad, gv1, d1, gf=gf1, center=c1, relu_out=True)

    samp1 = jax.random.permutation(jax.random.key(101), V)[:V // 8]
    nip1 = ni1[:, samp1, :8].reshape(-1)
    c2, s2 = _maxpool(_sc_gather(fm1, nip1).reshape(-1, 8, 64), w2, b2)
    verts1 = vertices[:, samp1, :]
    vertices_anchor = verts1
    V1 = V // 8

    ni2 = _knn(verts1, NB + 1)[:, :, 1:]
    ni2f = ni2.reshape(-1)
    v1pad = _pad16(verts1.reshape(-1, 3))
    gv2 = _sc_gather(v1pad, ni2f).reshape(bs * V1, NB, 16)

    gf2 = _sc_gather(s2, ni2f).reshape(bs * V1, NB, 128)
    c3, s3 = _agg(v1pad, gv2, d2, gf=gf2, center=c2, relu_out=True,
                  wl=w3, bl=b3)
    gf3 = _sc_gather(s3, ni2f).reshape(bs * V1, NB, 256)
    fm3 = _agg(v1pad, gv2, d3, gf=gf3, center=c3, relu_out=True)

    samp2 = jax.random.permutation(jax.random.key(202), V1)[:V1 // 8]
    nip2 = ni2[:, samp2, :8].reshape(-1)
    c4, s4 = _maxpool(_sc_gather(fm3, nip2).reshape(-1, 8, 256), w4, b4)
    verts2 = verts1[:, samp2, :]
    V2 = V1 // 8

    ni3 = _knn(verts2, NB + 1)[:, :, 1:]
    ni3f = ni3.reshape(-1)
    v2pad = _pad16(verts2.reshape(-1, 3))
    gv3 = _sc_gather(v2pad, ni3f).reshape(bs * V2, NB, 16)

    gf4 = _sc_gather(s4, ni3f).reshape(bs * V2, NB, 512)
    c5, s5 = _agg(v2pad, gv3, d4, gf=gf4, center=c4, relu_out=True,
                  wl=w5, bl=b5)
    gf5 = _sc_gather(s5, ni3f).reshape(bs * V2, NB, 512)
    fm5 = _agg(v2pad, gv3, d5, gf=gf5, center=c5, relu_out=True)

    samp3 = jax.random.permutation(jax.random.key(303), V2)[:V2 // 8]
    nip3 = ni3[:, samp3, :8].reshape(-1)
    c6, s6 = _maxpool(_sc_gather(fm5, nip3).reshape(-1, 8, 512), w6, b6)
    verts3 = verts2[:, samp3, :]
    V3 = V2 // 8

    ni4 = _knn(verts3, 4)[:, :, 1:]
    ni4f = ni4.reshape(-1)
    v3pad = _pad16(verts3.reshape(-1, 3))
    gv4 = _gather_padded(v3pad, ni4f).reshape(bs * V3, 3, 16)

    gf6 = _gather_padded(s6, ni4f).reshape(bs * V3, 3, 1024)
    fg = _agg(v3pad, gv4, d6, gf=gf6, center=c6,
              reduce_rows=True)
    return fg.reshape(bs, 1, 1024), vertices_anchor


# speedup vs baseline: 12.0293x; 1.1682x over previous
import functools
import math

import jax
import jax.numpy as jnp
from jax import lax
from jax.experimental import pallas as pl
from jax.experimental.pallas import tpu as pltpu
from jax.experimental.pallas import tpu_sc as plsc

_NC = 2
_NS = 16
_NW = _NC * _NS


def _knn_kernel(K, V, BR):
    def kf(v_ref, vt_ref, o_ref):
        b = pl.program_id(0)
        v = v_ref[0]
        vt = vt_ref[0]
        qr = jnp.sum(v * v, axis=1, keepdims=True)
        qc = jnp.sum(vt * vt, axis=0, keepdims=True)
        inner = (v[:, 0:1] * vt[0:1, :] + v[:, 1:2] * vt[1:2, :]
                 + v[:, 2:3] * vt[2:3, :])
        dist = -2.0 * inner + qc + qr
        iota = lax.broadcasted_iota(jnp.int32, (BR, V), 1)
        cols = []
        for _ in range(K):
            m = jnp.min(dist, axis=1, keepdims=True)
            sel = jnp.min(jnp.where(dist == m, iota, V), axis=1,
                          keepdims=True)
            cols.append(sel)
            dist = jnp.where(iota == sel, jnp.inf, dist)
        o_ref[0] = jnp.concatenate(cols, axis=1) + b * V
    return kf


def _knn(vertices, K):
    bs, V, _ = vertices.shape
    vt = jnp.swapaxes(vertices, 1, 2)
    BR = min(V, 256)
    nb = V // BR
    return pl.pallas_call(
        _knn_kernel(K, V, BR),
        grid=(bs, nb),
        in_specs=[
            pl.BlockSpec((1, BR, 3), lambda b, r: (b, r, 0)),
            pl.BlockSpec((1, 3, V), lambda b, r: (b, 0, 0)),
        ],
        out_specs=pl.BlockSpec((1, BR, K), lambda b, r: (b, r, 0)),
        out_shape=jax.ShapeDtypeStruct((bs, V, K), jnp.int32),
    )(vertices, vt)


def _sc_gather(table, idx2):
    NB, R = idx2.shape
    T, D = table.shape
    assert R % (8 * _NW) == 0
    vpw = R // _NW
    chunk = None
    for c in (128, 64, 32, 16, 8):
        if vpw % c == 0 and 2 * c * D * 4 <= 262144:
            chunk = c
            break
    nc = vpw // chunk
    m_iter = NB * nc
    pipelined = m_iter > 1 and m_iter % 2 == 0
    mesh = plsc.VectorSubcoreMesh(core_axis_name="c", subcore_axis_name="s",
                                  num_cores=_NC, num_subcores=_NS)

    @functools.partial(
        pl.kernel,
        out_type=jax.ShapeDtypeStruct((R, NB * D), jnp.float32),
        mesh=mesh,
        scratch_types=[
            pltpu.VMEM((NB, vpw), jnp.int32),
            pltpu.VMEM((chunk, D), jnp.float32),
            pltpu.VMEM((chunk, D), jnp.float32),
            pltpu.SemaphoreType.DMA,
            pltpu.SemaphoreType.DMA,
        ],
        compiler_params=pltpu.CompilerParams(use_tc_tiling_on_sc=False),
    )
    def k(table_hbm, idx_hbm, out_hbm, idx_v, buf0, buf1, sem0, sem1):
        wid = lax.axis_index("s") * _NC + lax.axis_index("c")
        base = wid * vpw
        pltpu.sync_copy(idx_hbm.at[:, pl.ds(base, vpw)], idx_v)

        def src(i):
            n = i // nc
            c = i % nc
            return table_hbm.at[idx_v.at[n, pl.ds(c * chunk, chunk)]]

        def dst(i):
            n = i // nc
            c = i % nc
            return out_hbm.at[pl.ds(base + c * chunk, chunk),
                              pl.ds(n * D, D)]

        if not pipelined:
            def body(i, carry):
                pltpu.async_copy(src(i), buf0, sem0).wait()
                pltpu.sync_copy(buf0, dst(i))
                return carry
            lax.fori_loop(0, m_iter, body, 0)
        else:
            pltpu.async_copy(src(0), buf0, sem0)

            def body(j, carry):
                i0 = 2 * j
                pltpu.make_async_copy(src(i0), buf0, sem0).wait()
                pltpu.async_copy(src(i0 + 1), buf1, sem1)
                pltpu.sync_copy(buf0, dst(i0))
                pltpu.make_async_copy(src(i0 + 1), buf1, sem1).wait()

                @pl.when(i0 + 2 < m_iter)
                def _():
                    pltpu.async_copy(src(i0 + 2), buf0, sem0)

                pltpu.sync_copy(buf1, dst(i0 + 1))
                return carry

            lax.fori_loop(0, m_iter // 2, body, 0)

    return k(table, idx2)


def _linear_epilogue(fm, wl_ref, bl_ref, oc_ref, os_ref):
    fo = lax.dot_general(fm, wl_ref[...], (((1,), (0,)), ((), ())),
                         preferred_element_type=jnp.float32) + bl_ref[...]
    half = fo.shape[1] // 2
    oc_ref[...] = fo[:, :half]
    os_ref[...] = fo[:, half:]


def _agg(x, gv, dmat, gf=None, center=None, w0=None, relu_out=False,
         reduce_rows=False, wl=None, bl=None, rows=None):
    R = rows if rows is not None else x.shape[0]
    N = gv.shape[1] // 16
    C = dmat.shape[1]
    if reduce_rows:
        BR = R
        nb = 1
    else:
        BR = None
        for c in (256, 128, 64, 32, 16, 8, 4):
            if R % c == 0 and c * N * (16 + max(C, 128)) * 4 <= 6 * 2**20:
                BR = c
                break
        nb = R // BR

    def kf(*refs):
        it = iter(refs)
        x_ref = next(it)
        gv_ref = next(it)
        d_ref = next(it)
        gf_ref = next(it) if gf is not None else None
        c_ref = next(it) if center is not None else None
        w0_ref = next(it) if w0 is not None else None
        wl_ref = next(it) if wl is not None else None
        bl_ref = next(it) if wl is not None else None
        o_refs = list(it)
        dm = d_ref[...]
        xv = x_ref[...]
        acc = None
        for n in range(N):
            disp = gv_ref[:, 16 * n:16 * (n + 1)] - xv
            th = jnp.maximum(
                lax.dot_general(disp, dm, (((1,), (0,)), ((), ())),
                                preferred_element_type=jnp.float32), 0.0)
            if gf_ref is not None:
                th = th * gf_ref[:, C * n:C * (n + 1)]
            acc = th if acc is None else jnp.maximum(acc, th)
        if w0_ref is not None:
            acc = acc * w0_ref[...]
        if c_ref is not None:
            acc = c_ref[...] + acc
        if relu_out:
            acc = jnp.maximum(acc, 0.0)
        if reduce_rows:
            rws = [jnp.max(acc[4 * g:4 * g + 4, :], axis=0, keepdims=True)
                   for g in range(R // 4)]
            o_refs[0][...] = jnp.concatenate(rws, axis=0)
        elif wl is not None:
            _linear_epilogue(acc, wl_ref, bl_ref, o_refs[0], o_refs[1])
        else:
            o_refs[0][...] = acc

    in_specs = [
        pl.BlockSpec((BR, 16), lambda r: (r, 0)),
        pl.BlockSpec((BR, N * 16), lambda r: (r, 0)),
        pl.BlockSpec((16, C), lambda r: (0, 0)),
    ]
    args = [x, gv, jnp.pad(dmat, ((0, 13), (0, 0)))]
    if gf is not None:
        in_specs.append(pl.BlockSpec((BR, N * C), lambda r: (r, 0)))
        args.append(gf)
    if center is not None:
        in_specs.append(pl.BlockSpec((BR, C), lambda r: (r, 0)))
        args.append(center)
    if w0 is not None:
        in_specs.append(pl.BlockSpec((1, C), lambda r: (0, 0)))
        args.append(w0)
    if wl is not None:
        Co = wl.shape[1] // 2
        in_specs.append(pl.BlockSpec(wl.shape, lambda r: (0, 0)))
        in_specs.append(pl.BlockSpec((1, 2 * Co), lambda r: (0, 0)))
        args += [wl, bl.reshape(1, -1)]
        out_spec = [pl.BlockSpec((BR, Co), lambda r: (r, 0))] * 2
        out_shape = [jax.ShapeDtypeStruct((R, Co), jnp.float32)] * 2
    elif reduce_rows:
        out_spec = pl.BlockSpec((R // 4, C), lambda r: (0, 0))
        out_shape = jax.ShapeDtypeStruct((R // 4, C), jnp.float32)
    else:
        out_spec = pl.BlockSpec((BR, C), lambda r: (r, 0))
        out_shape = jax.ShapeDtypeStruct((R, C), jnp.float32)
    return pl.pallas_call(
        kf,
        grid=(nb,),
        in_specs=in_specs,
        out_specs=out_spec,
        out_shape=out_shape,
    )(*args)


def _maxpool(gp, C, wl, bl, rows=None):
    R = rows if rows is not None else gp.shape[0]
    N = gp.shape[1] // C
    BR = None
    for c in (256, 128, 64, 32, 16, 8, 4):
        if R % c == 0 and c * N * max(C, 128) * 4 <= 6 * 2**20:
            BR = c
            break
    nb = R // BR
    Co = wl.shape[1] // 2

    def kf(g_ref, wl_ref, bl_ref, oc_ref, os_ref):
        acc = g_ref[:, 0:C]
        for n in range(1, N):
            acc = jnp.maximum(acc, g_ref[:, C * n:C * (n + 1)])
        _linear_epilogue(acc, wl_ref, bl_ref, oc_ref, os_ref)

    return pl.pallas_call(
        kf,
        grid=(nb,),
        in_specs=[
            pl.BlockSpec((BR, N * C), lambda r: (r, 0)),
            pl.BlockSpec(wl.shape, lambda r: (0, 0)),
            pl.BlockSpec((1, 2 * Co), lambda r: (0, 0)),
        ],
        out_specs=[pl.BlockSpec((BR, Co), lambda r: (r, 0))] * 2,
        out_shape=[jax.ShapeDtypeStruct((R, Co), jnp.float32)] * 2,
    )(gp, wl, bl.reshape(1, -1))


def _pad16(v):
    return jnp.pad(v, ((0, 0), (0, 13)))


def _gather_padded(table, idx2):
    NB, R = idx2.shape
    Rp = ((R + 255) // 256) * 256
    if Rp != R:
        idx2 = jnp.concatenate(
            [idx2, jnp.zeros((NB, Rp - R), jnp.int32)], axis=1)
    return _sc_gather(table, idx2)


def _nmajor(ni):
    bs, V, NB = ni.shape
    return jnp.swapaxes(ni.reshape(bs * V, NB), 0, 1)


def kernel(vertices, w0, d0, w1, b1, d1, w2, b2, d2, w3, b3, d3,
           w4, b4, d4, w5, b5, d5, w6, b6, d6):
    bs, V, _ = vertices.shape
    NB = 20

    ni1 = _knn(vertices, NB + 1)[:, :, 1:]
    ni1t = _nmajor(ni1)
    vpad = _pad16(vertices.reshape(-1, 3))
    gv1 = _sc_gather(vpad, ni1t)

    c1, s1 = _agg(vpad, gv1, d0, w0=w0.reshape(1, -1), relu_out=True,
                  wl=w1, bl=b1)
    gf1 = _sc_gather(s1, ni1t)
    fm1 = _agg(vpad, gv1, d1, gf=gf1, center=c1, relu_out=True)

    samp1 = jax.random.permutation(jax.random.key(101), V)[:V // 8]
    nip1 = _nmajor(ni1[:, samp1, :8])
    c2, s2 = _maxpool(_sc_gather(fm1, nip1), 64, w2, b2)
    verts1 = vertices[:, samp1, :]
    vertices_anchor = verts1
    V1 = V // 8

    ni2 = _knn(verts1, NB + 1)[:, :, 1:]
    ni2t = _nmajor(ni2)
    v1pad = _pad16(verts1.reshape(-1, 3))
    gv2 = _sc_gather(v1pad, ni2t)

    gf2 = _sc_gather(s2, ni2t)
    c3, s3 = _agg(v1pad, gv2, d2, gf=gf2, center=c2, relu_out=True,
                  wl=w3, bl=b3)
    gf3 = _sc_gather(s3, ni2t)
    fm3 = _agg(v1pad, gv2, d3, gf=gf3, center=c3, relu_out=True)

    samp2 = jax.random.permutation(jax.random.key(202), V1)[:V1 // 8]
    nip2 = _nmajor(ni2[:, samp2, :8])
    c4, s4 = _maxpool(_sc_gather(fm3, nip2), 256, w4, b4)
    verts2 = verts1[:, samp2, :]
    V2 = V1 // 8

    ni3 = _knn(verts2, NB + 1)[:, :, 1:]
    ni3t = _nmajor(ni3)
    v2pad = _pad16(verts2.reshape(-1, 3))
    gv3 = _sc_gather(v2pad, ni3t)

    gf4 = _sc_gather(s4, ni3t)
    c5, s5 = _agg(v2pad, gv3, d4, gf=gf4, center=c4, relu_out=True,
                  wl=w5, bl=b5)
    gf5 = _sc_gather(s5, ni3t)
    fm5 = _agg(v2pad, gv3, d5, gf=gf5, center=c5, relu_out=True)

    samp3 = jax.random.permutation(jax.random.key(303), V2)[:V2 // 8]
    nip3 = _nmajor(ni3[:, samp3, :8])
    c6, s6 = _maxpool(_gather_padded(fm5, nip3), 512, w6, b6, rows=bs * 4)
    verts3 = verts2[:, samp3, :]
    V3 = V2 // 8

    ni4 = _knn(verts3, 4)[:, :, 1:]
    ni4t = _nmajor(ni4)
    v3pad = _pad16(verts3.reshape(-1, 3))
    gv4 = _gather_padded(v3pad, ni4t)

    gf6 = _gather_padded(s6, ni4t)
    fg = _agg(v3pad, gv4, d6, gf=gf6, center=c6,
              reduce_rows=True, rows=bs * V3)
    return fg.reshape(bs, 1, 1024), vertices_anchor


# speedup vs baseline: 14.5989x; 1.2136x over previous
import functools
import math

import jax
import jax.numpy as jnp
from jax import lax
from jax.experimental import pallas as pl
from jax.experimental.pallas import tpu as pltpu
from jax.experimental.pallas import tpu_sc as plsc

_NC = 2
_NS = 16
_NW = _NC * _NS


def _knn_kernel(K, V, BR):
    def kf(v_ref, vt_ref, o_ref):
        b = pl.program_id(0)
        v = v_ref[0]
        vt = vt_ref[0]
        qr = jnp.sum(v * v, axis=1, keepdims=True)
        qc = jnp.sum(vt * vt, axis=0, keepdims=True)
        inner = (v[:, 0:1] * vt[0:1, :] + v[:, 1:2] * vt[1:2, :]
                 + v[:, 2:3] * vt[2:3, :])
        dist = -2.0 * inner + qc + qr
        iota = lax.broadcasted_iota(jnp.int32, (BR, V), 1)
        cols = []
        for _ in range(K):
            sel = jnp.argmin(dist, axis=1).reshape(BR, 1)
            cols.append(sel)
            dist = jnp.where(iota == sel, jnp.inf, dist)
        o_ref[0] = jnp.concatenate(cols, axis=1) + b * V
    return kf


def _knn(vertices, K):
    bs, V, _ = vertices.shape
    vt = jnp.swapaxes(vertices, 1, 2)
    BR = min(V, 256)
    nb = V // BR
    return pl.pallas_call(
        _knn_kernel(K, V, BR),
        grid=(bs, nb),
        in_specs=[
            pl.BlockSpec((1, BR, 3), lambda b, r: (b, r, 0)),
            pl.BlockSpec((1, 3, V), lambda b, r: (b, 0, 0)),
        ],
        out_specs=pl.BlockSpec((1, BR, K), lambda b, r: (b, r, 0)),
        out_shape=jax.ShapeDtypeStruct((bs, V, K), jnp.int32),
    )(vertices, vt)


def _sc_gather(table, idx2):
    NB, R = idx2.shape
    T, D = table.shape
    assert R % (8 * _NW) == 0
    vpw = R // _NW
    chunk = None
    for c in (128, 64, 32, 16, 8):
        if vpw % c == 0 and 2 * c * D * 4 <= 262144:
            chunk = c
            break
    nc = vpw // chunk
    m_iter = NB * nc
    NBUF = 4
    pipelined = m_iter % NBUF == 0
    mesh = plsc.VectorSubcoreMesh(core_axis_name="c", subcore_axis_name="s",
                                  num_cores=_NC, num_subcores=_NS)

    @functools.partial(
        pl.kernel,
        out_type=jax.ShapeDtypeStruct((R, NB * D), jnp.float32),
        mesh=mesh,
        scratch_types=[
            pltpu.VMEM((NB, vpw), jnp.int32),
        ] + [pltpu.VMEM((chunk, D), jnp.float32)] * NBUF
          + [pltpu.SemaphoreType.DMA] * NBUF,
        compiler_params=pltpu.CompilerParams(use_tc_tiling_on_sc=False),
    )
    def k(table_hbm, idx_hbm, out_hbm, idx_v, *rest):
        bufs = rest[:NBUF]
        sems = rest[NBUF:]
        wid = lax.axis_index("s") * _NC + lax.axis_index("c")
        base = wid * vpw
        pltpu.sync_copy(idx_hbm.at[:, pl.ds(base, vpw)], idx_v)

        def src(i):
            n = i // nc
            c = i % nc
            return table_hbm.at[idx_v.at[n, pl.ds(c * chunk, chunk)]]

        def dst(i):
            n = i // nc
            c = i % nc
            return out_hbm.at[pl.ds(base + c * chunk, chunk),
                              pl.ds(n * D, D)]

        if not pipelined:
            def body(i, carry):
                pltpu.async_copy(src(i), bufs[0], sems[0]).wait()
                pltpu.sync_copy(bufs[0], dst(i))
                return carry
            lax.fori_loop(0, m_iter, body, 0)
        else:
            for b in range(min(NBUF - 1, m_iter)):
                pltpu.async_copy(src(b), bufs[b], sems[b])

            def body(j, carry):
                for b in range(NBUF):
                    i = NBUF * j + b
                    pltpu.make_async_copy(src(i), bufs[b], sems[b]).wait()
                    nxt = i + NBUF - 1
                    bn = (b + NBUF - 1) % NBUF

                    @pl.when(nxt < m_iter)
                    def _(nxt=nxt, bn=bn):
                        pltpu.async_copy(src(nxt), bufs[bn], sems[bn])

                    pltpu.sync_copy(bufs[b], dst(i))
                return carry

            lax.fori_loop(0, m_iter // NBUF, body, 0)

    return k(table, idx2)


def _linear_epilogue(fm, wl_ref, bl_ref, oc_ref, os_ref):
    fo = lax.dot_general(fm, wl_ref[...], (((1,), (0,)), ((), ())),
                         preferred_element_type=jnp.float32) + bl_ref[...]
    half = fo.shape[1] // 2
    oc_ref[...] = fo[:, :half]
    os_ref[...] = fo[:, half:]


def _agg(x, gv, dmat, gf=None, center=None, w0=None, relu_out=False,
         reduce_rows=False, wl=None, bl=None, rows=None):
    R = rows if rows is not None else x.shape[0]
    N = gv.shape[1] // 16
    C = dmat.shape[1]
    if reduce_rows:
        BR = R
        nb = 1
    else:
        BR = None
        for c in (256, 128, 64, 32, 16, 8, 4):
            if R % c == 0 and c * N * (16 + max(C, 128)) * 4 <= 6 * 2**20:
                BR = c
                break
        nb = R // BR

    def kf(*refs):
        it = iter(refs)
        x_ref = next(it)
        gv_ref = next(it)
        d_ref = next(it)
        gf_ref = next(it) if gf is not None else None
        c_ref = next(it) if center is not None else None
        w0_ref = next(it) if w0 is not None else None
        wl_ref = next(it) if wl is not None else None
        bl_ref = next(it) if wl is not None else None
        o_refs = list(it)
        dm = d_ref[...]
        xv = x_ref[...]
        acc = None
        for n in range(N):
            disp = gv_ref[:, 16 * n:16 * (n + 1)] - xv
            th = jnp.maximum(
                lax.dot_general(disp, dm, (((1,), (0,)), ((), ())),
                                preferred_element_type=jnp.float32), 0.0)
            if gf_ref is not None:
                th = th * gf_ref[:, C * n:C * (n + 1)]
            acc = th if acc is None else jnp.maximum(acc, th)
        if w0_ref is not None:
            acc = acc * w0_ref[...]
        if c_ref is not None:
            acc = c_ref[...] + acc
        if relu_out:
            acc = jnp.maximum(acc, 0.0)
        if reduce_rows:
            rws = [jnp.max(acc[4 * g:4 * g + 4, :], axis=0, keepdims=True)
                   for g in range(R // 4)]
            o_refs[0][...] = jnp.concatenate(rws, axis=0)
        elif wl is not None:
            _linear_epilogue(acc, wl_ref, bl_ref, o_refs[0], o_refs[1])
        else:
            o_refs[0][...] = acc

    in_specs = [
        pl.BlockSpec((BR, 16), lambda r: (r, 0)),
        pl.BlockSpec((BR, N * 16), lambda r: (r, 0)),
        pl.BlockSpec((16, C), lambda r: (0, 0)),
    ]
    args = [x, gv, jnp.pad(dmat, ((0, 13), (0, 0)))]
    if gf is not None:
        in_specs.append(pl.BlockSpec((BR, N * C), lambda r: (r, 0)))
        args.append(gf)
    if center is not None:
        in_specs.append(pl.BlockSpec((BR, C), lambda r: (r, 0)))
        args.append(center)
    if w0 is not None:
        in_specs.append(pl.BlockSpec((1, C), lambda r: (0, 0)))
        args.append(w0)
    if wl is not None:
        Co = wl.shape[1] // 2
        in_specs.append(pl.BlockSpec(wl.shape, lambda r: (0, 0)))
        in_specs.append(pl.BlockSpec((1, 2 * Co), lambda r: (0, 0)))
        args += [wl, bl.reshape(1, -1)]
        out_spec = [pl.BlockSpec((BR, Co), lambda r: (r, 0))] * 2
        out_shape = [jax.ShapeDtypeStruct((R, Co), jnp.float32)] * 2
    elif reduce_rows:
        out_spec = pl.BlockSpec((R // 4, C), lambda r: (0, 0))
        out_shape = jax.ShapeDtypeStruct((R // 4, C), jnp.float32)
    else:
        out_spec = pl.BlockSpec((BR, C), lambda r: (r, 0))
        out_shape = jax.ShapeDtypeStruct((R, C), jnp.float32)
    return pl.pallas_call(
        kf,
        grid=(nb,),
        in_specs=in_specs,
        out_specs=out_spec,
        out_shape=out_shape,
    )(*args)


def _maxpool(gp, C, wl, bl, rows=None):
    R = rows if rows is not None else gp.shape[0]
    N = gp.shape[1] // C
    BR = None
    for c in (256, 128, 64, 32, 16, 8, 4):
        if R % c == 0 and c * N * max(C, 128) * 4 <= 6 * 2**20:
            BR = c
            break
    nb = R // BR
    Co = wl.shape[1] // 2

    def kf(g_ref, wl_ref, bl_ref, oc_ref, os_ref):
        acc = g_ref[:, 0:C]
        for n in range(1, N):
            acc = jnp.maximum(acc, g_ref[:, C * n:C * (n + 1)])
        _linear_epilogue(acc, wl_ref, bl_ref, oc_ref, os_ref)

    return pl.pallas_call(
        kf,
        grid=(nb,),
        in_specs=[
            pl.BlockSpec((BR, N * C), lambda r: (r, 0)),
            pl.BlockSpec(wl.shape, lambda r: (0, 0)),
            pl.BlockSpec((1, 2 * Co), lambda r: (0, 0)),
        ],
        out_specs=[pl.BlockSpec((BR, Co), lambda r: (r, 0))] * 2,
        out_shape=[jax.ShapeDtypeStruct((R, Co), jnp.float32)] * 2,
    )(gp, wl, bl.reshape(1, -1))


def _pad16(v):
    return jnp.pad(v, ((0, 0), (0, 13)))


def _gather_padded(table, idx2):
    NB, R = idx2.shape
    Rp = ((R + 255) // 256) * 256
    if Rp != R:
        idx2 = jnp.concatenate(
            [idx2, jnp.zeros((NB, Rp - R), jnp.int32)], axis=1)
    return _sc_gather(table, idx2)


def _nmajor(ni):
    bs, V, NB = ni.shape
    return jnp.swapaxes(ni.reshape(bs * V, NB), 0, 1)


def kernel(vertices, w0, d0, w1, b1, d1, w2, b2, d2, w3, b3, d3,
           w4, b4, d4, w5, b5, d5, w6, b6, d6):
    bs, V, _ = vertices.shape
    NB = 20

    ni1 = _knn(vertices, NB + 1)[:, :, 1:]
    ni1t = _nmajor(ni1)
    vpad = _pad16(vertices.reshape(-1, 3))
    gv1 = _sc_gather(vpad, ni1t)

    c1, s1 = _agg(vpad, gv1, d0, w0=w0.reshape(1, -1), relu_out=True,
                  wl=w1, bl=b1)
    gf1 = _sc_gather(s1, ni1t)
    fm1 = _agg(vpad, gv1, d1, gf=gf1, center=c1, relu_out=True)

    samp1 = jax.random.permutation(jax.random.key(101), V)[:V // 8]
    nip1 = _nmajor(ni1[:, samp1, :8])
    c2, s2 = _maxpool(_sc_gather(fm1, nip1), 64, w2, b2)
    verts1 = vertices[:, samp1, :]
    vertices_anchor = verts1
    V1 = V // 8

    ni2 = _knn(verts1, NB + 1)[:, :, 1:]
    ni2t = _nmajor(ni2)
    v1pad = _pad16(verts1.reshape(-1, 3))
    gv2 = _sc_gather(v1pad, ni2t)

    gf2 = _sc_gather(s2, ni2t)
    c3, s3 = _agg(v1pad, gv2, d2, gf=gf2, center=c2, relu_out=True,
                  wl=w3, bl=b3)
    gf3 = _sc_gather(s3, ni2t)
    fm3 = _agg(v1pad, gv2, d3, gf=gf3, center=c3, relu_out=True)

    samp2 = jax.random.permutation(jax.random.key(202), V1)[:V1 // 8]
    nip2 = _nmajor(ni2[:, samp2, :8])
    c4, s4 = _maxpool(_sc_gather(fm3, nip2), 256, w4, b4)
    verts2 = verts1[:, samp2, :]
    V2 = V1 // 8

    ni3 = _knn(verts2, NB + 1)[:, :, 1:]
    ni3t = _nmajor(ni3)
    v2pad = _pad16(verts2.reshape(-1, 3))
    gv3 = _sc_gather(v2pad, ni3t)

    gf4 = _sc_gather(s4, ni3t)
    c5, s5 = _agg(v2pad, gv3, d4, gf=gf4, center=c4, relu_out=True,
                  wl=w5, bl=b5)
    gf5 = _sc_gather(s5, ni3t)
    fm5 = _agg(v2pad, gv3, d5, gf=gf5, center=c5, relu_out=True)

    samp3 = jax.random.permutation(jax.random.key(303), V2)[:V2 // 8]
    nip3 = _nmajor(ni3[:, samp3, :8])
    c6, s6 = _maxpool(_gather_padded(fm5, nip3), 512, w6, b6, rows=bs * 4)
    verts3 = verts2[:, samp3, :]
    V3 = V2 // 8

    ni4 = _knn(verts3, 4)[:, :, 1:]
    ni4t = _nmajor(ni4)
    v3pad = _pad16(verts3.reshape(-1, 3))
    gv4 = _gather_padded(v3pad, ni4t)

    gf6 = _gather_padded(s6, ni4t)
    fg = _agg(v3pad, gv4, d6, gf=gf6, center=c6,
              reduce_rows=True, rows=bs * V3)
    return fg.reshape(bs, 1, 1024), vertices_anchor


# speedup vs baseline: 14.6549x; 1.0038x over previous
import functools
import math

import jax
import jax.numpy as jnp
from jax import lax
from jax.experimental import pallas as pl
from jax.experimental.pallas import tpu as pltpu
from jax.experimental.pallas import tpu_sc as plsc

_NC = 2
_NS = 16
_NW = _NC * _NS


def _knn_kernel(K, V, BR):
    def kf(v_ref, vt_ref, o_ref):
        b = pl.program_id(0)
        v = v_ref[0]
        vt = vt_ref[0]
        qr = jnp.sum(v * v, axis=1, keepdims=True)
        qc = jnp.sum(vt * vt, axis=0, keepdims=True)
        inner = (v[:, 0:1] * vt[0:1, :] + v[:, 1:2] * vt[1:2, :]
                 + v[:, 2:3] * vt[2:3, :])
        dist = -2.0 * inner + qc + qr
        iota = lax.broadcasted_iota(jnp.int32, (BR, V), 1)
        cols = []
        for _ in range(K):
            sel = jnp.argmin(dist, axis=1).reshape(BR, 1)
            cols.append(sel)
            dist = jnp.where(iota == sel, jnp.inf, dist)
        o_ref[0] = jnp.concatenate(cols, axis=1) + b * V
    return kf


def _knn(vertices, K):
    bs, V, _ = vertices.shape
    vt = jnp.swapaxes(vertices, 1, 2)
    BR = min(V, 512)
    nb = V // BR
    return pl.pallas_call(
        _knn_kernel(K, V, BR),
        grid=(bs, nb),
        in_specs=[
            pl.BlockSpec((1, BR, 3), lambda b, r: (b, r, 0)),
            pl.BlockSpec((1, 3, V), lambda b, r: (b, 0, 0)),
        ],
        out_specs=pl.BlockSpec((1, BR, K), lambda b, r: (b, r, 0)),
        out_shape=jax.ShapeDtypeStruct((bs, V, K), jnp.int32),
    )(vertices, vt)


def _sc_gather(table, idx2):
    NB, R = idx2.shape
    T, D = table.shape
    assert R % (8 * _NW) == 0
    vpw = R // _NW
    chunk = None
    for c in (128, 64, 32, 16, 8):
        if vpw % c == 0 and 2 * c * D * 4 <= 262144:
            chunk = c
            break
    nc = vpw // chunk
    m_iter = NB * nc
    NBUF = 4
    pipelined = m_iter % NBUF == 0
    mesh = plsc.VectorSubcoreMesh(core_axis_name="c", subcore_axis_name="s",
                                  num_cores=_NC, num_subcores=_NS)

    @functools.partial(
        pl.kernel,
        out_type=jax.ShapeDtypeStruct((R, NB * D), jnp.float32),
        mesh=mesh,
        scratch_types=[
            pltpu.VMEM((NB, vpw), jnp.int32),
        ] + [pltpu.VMEM((chunk, D), jnp.float32)] * NBUF
          + [pltpu.SemaphoreType.DMA] * NBUF,
        compiler_params=pltpu.CompilerParams(use_tc_tiling_on_sc=False),
    )
    def k(table_hbm, idx_hbm, out_hbm, idx_v, *rest):
        bufs = rest[:NBUF]
        sems = rest[NBUF:]
        wid = lax.axis_index("s") * _NC + lax.axis_index("c")
        base = wid * vpw
        pltpu.sync_copy(idx_hbm.at[:, pl.ds(base, vpw)], idx_v)

        def src(i):
            n = i // nc
            c = i % nc
            return table_hbm.at[idx_v.at[n, pl.ds(c * chunk, chunk)]]

        def dst(i):
            n = i // nc
            c = i % nc
            return out_hbm.at[pl.ds(base + c * chunk, chunk),
                              pl.ds(n * D, D)]

        if not pipelined:
            def body(i, carry):
                pltpu.async_copy(src(i), bufs[0], sems[0]).wait()
                pltpu.sync_copy(bufs[0], dst(i))
                return carry
            lax.fori_loop(0, m_iter, body, 0)
        else:
            for b in range(min(NBUF - 1, m_iter)):
                pltpu.async_copy(src(b), bufs[b], sems[b])

            def body(j, carry):
                for b in range(NBUF):
                    i = NBUF * j + b
                    pltpu.make_async_copy(src(i), bufs[b], sems[b]).wait()
                    nxt = i + NBUF - 1
                    bn = (b + NBUF - 1) % NBUF

                    @pl.when(nxt < m_iter)
                    def _(nxt=nxt, bn=bn):
                        pltpu.async_copy(src(nxt), bufs[bn], sems[bn])

                    pltpu.sync_copy(bufs[b], dst(i))
                return carry

            lax.fori_loop(0, m_iter // NBUF, body, 0)

    return k(table, idx2)


def _linear_epilogue(fm, wl_ref, bl_ref, oc_ref, os_ref):
    fo = lax.dot_general(fm, wl_ref[...], (((1,), (0,)), ((), ())),
                         preferred_element_type=jnp.float32) + bl_ref[...]
    half = fo.shape[1] // 2
    oc_ref[...] = fo[:, :half]
    os_ref[...] = fo[:, half:]


def _agg(x, gv, dmat, gf=None, center=None, w0=None, relu_out=False,
         reduce_rows=False, wl=None, bl=None, rows=None):
    R = rows if rows is not None else x.shape[0]
    N = gv.shape[1] // 16
    C = dmat.shape[1]
    if reduce_rows:
        BR = R
        nb = 1
    else:
        BR = None
        for c in (256, 128, 64, 32, 16, 8, 4):
            if R % c == 0 and c * N * (16 + max(C, 128)) * 4 <= 6 * 2**20:
                BR = c
                break
        nb = R // BR

    def kf(*refs):
        it = iter(refs)
        x_ref = next(it)
        gv_ref = next(it)
        d_ref = next(it)
        gf_ref = next(it) if gf is not None else None
        c_ref = next(it) if center is not None else None
        w0_ref = next(it) if w0 is not None else None
        wl_ref = next(it) if wl is not None else None
        bl_ref = next(it) if wl is not None else None
        o_refs = list(it)
        dm = d_ref[...]
        xv = x_ref[...]
        acc = None
        for n in range(N):
            disp = gv_ref[:, 16 * n:16 * (n + 1)] - xv
            th = jnp.maximum(
                lax.dot_general(disp, dm, (((1,), (0,)), ((), ())),
                                preferred_element_type=jnp.float32), 0.0)
            if gf_ref is not None:
                th = th * gf_ref[:, C * n:C * (n + 1)]
            acc = th if acc is None else jnp.maximum(acc, th)
        if w0_ref is not None:
            acc = acc * w0_ref[...]
        if c_ref is not None:
            acc = c_ref[...] + acc
        if relu_out:
            acc = jnp.maximum(acc, 0.0)
        if reduce_rows:
            rws = [jnp.max(acc[4 * g:4 * g + 4, :], axis=0, keepdims=True)
                   for g in range(R // 4)]
            o_refs[0][...] = jnp.concatenate(rws, axis=0)
        elif wl is not None:
            _linear_epilogue(acc, wl_ref, bl_ref, o_refs[0], o_refs[1])
        else:
            o_refs[0][...] = acc

    in_specs = [
        pl.BlockSpec((BR, 16), lambda r: (r, 0)),
        pl.BlockSpec((BR, N * 16), lambda r: (r, 0)),
        pl.BlockSpec((16, C), lambda r: (0, 0)),
    ]
    args = [x, gv, jnp.pad(dmat, ((0, 13), (0, 0)))]
    if gf is not None:
        in_specs.append(pl.BlockSpec((BR, N * C), lambda r: (r, 0)))
        args.append(gf)
    if center is not None:
        in_specs.append(pl.BlockSpec((BR, C), lambda r: (r, 0)))
        args.append(center)
    if w0 is not None:
        in_specs.append(pl.BlockSpec((1, C), lambda r: (0, 0)))
        args.append(w0)
    if wl is not None:
        Co = wl.shape[1] // 2
        in_specs.append(pl.BlockSpec(wl.shape, lambda r: (0, 0)))
        in_specs.append(pl.BlockSpec((1, 2 * Co), lambda r: (0, 0)))
        args += [wl, bl.reshape(1, -1)]
        out_spec = [pl.BlockSpec((BR, Co), lambda r: (r, 0))] * 2
        out_shape = [jax.ShapeDtypeStruct((R, Co), jnp.float32)] * 2
    elif reduce_rows:
        out_spec = pl.BlockSpec((R // 4, C), lambda r: (0, 0))
        out_shape = jax.ShapeDtypeStruct((R // 4, C), jnp.float32)
    else:
        out_spec = pl.BlockSpec((BR, C), lambda r: (r, 0))
        out_shape = jax.ShapeDtypeStruct((R, C), jnp.float32)
    return pl.pallas_call(
        kf,
        grid=(nb,),
        in_specs=in_specs,
        out_specs=out_spec,
        out_shape=out_shape,
    )(*args)


def _maxpool(gp, C, wl, bl, rows=None):
    R = rows if rows is not None else gp.shape[0]
    N = gp.shape[1] // C
    BR = None
    for c in (256, 128, 64, 32, 16, 8, 4):
        if R % c == 0 and c * N * max(C, 128) * 4 <= 6 * 2**20:
            BR = c
            break
    nb = R // BR
    Co = wl.shape[1] // 2

    def kf(g_ref, wl_ref, bl_ref, oc_ref, os_ref):
        acc = g_ref[:, 0:C]
        for n in range(1, N):
            acc = jnp.maximum(acc, g_ref[:, C * n:C * (n + 1)])
        _linear_epilogue(acc, wl_ref, bl_ref, oc_ref, os_ref)

    return pl.pallas_call(
        kf,
        grid=(nb,),
        in_specs=[
            pl.BlockSpec((BR, N * C), lambda r: (r, 0)),
            pl.BlockSpec(wl.shape, lambda r: (0, 0)),
            pl.BlockSpec((1, 2 * Co), lambda r: (0, 0)),
        ],
        out_specs=[pl.BlockSpec((BR, Co), lambda r: (r, 0))] * 2,
        out_shape=[jax.ShapeDtypeStruct((R, Co), jnp.float32)] * 2,
    )(gp, wl, bl.reshape(1, -1))


def _pad16(v):
    return jnp.pad(v, ((0, 0), (0, 13)))


def _gather_padded(table, idx2):
    NB, R = idx2.shape
    Rp = ((R + 255) // 256) * 256
    if Rp != R:
        idx2 = jnp.concatenate(
            [idx2, jnp.zeros((NB, Rp - R), jnp.int32)], axis=1)
    return _sc_gather(table, idx2)


def _nmajor(ni):
    bs, V, NB = ni.shape
    return jnp.swapaxes(ni.reshape(bs * V, NB), 0, 1)


def kernel(vertices, w0, d0, w1, b1, d1, w2, b2, d2, w3, b3, d3,
           w4, b4, d4, w5, b5, d5, w6, b6, d6):
    bs, V, _ = vertices.shape
    NB = 20

    ni1 = _knn(vertices, NB + 1)[:, :, 1:]
    ni1t = _nmajor(ni1)
    vpad = _pad16(vertices.reshape(-1, 3))
    gv1 = _sc_gather(vpad, ni1t)

    c1, s1 = _agg(vpad, gv1, d0, w0=w0.reshape(1, -1), relu_out=True,
                  wl=w1, bl=b1)
    gf1 = _sc_gather(s1, ni1t)
    fm1 = _agg(vpad, gv1, d1, gf=gf1, center=c1, relu_out=True)

    samp1 = jax.random.permutation(jax.random.key(101), V)[:V // 8]
    nip1 = _nmajor(ni1[:, samp1, :8])
    c2, s2 = _maxpool(_sc_gather(fm1, nip1), 64, w2, b2)
    verts1 = vertices[:, samp1, :]
    vertices_anchor = verts1
    V1 = V // 8

    ni2 = _knn(verts1, NB + 1)[:, :, 1:]
    ni2t = _nmajor(ni2)
    v1pad = _pad16(verts1.reshape(-1, 3))
    gv2 = _sc_gather(v1pad, ni2t)

    gf2 = _sc_gather(s2, ni2t)
    c3, s3 = _agg(v1pad, gv2, d2, gf=gf2, center=c2, relu_out=True,
                  wl=w3, bl=b3)
    gf3 = _sc_gather(s3, ni2t)
    fm3 = _agg(v1pad, gv2, d3, gf=gf3, center=c3, relu_out=True)

    samp2 = jax.random.permutation(jax.random.key(202), V1)[:V1 // 8]
    nip2 = _nmajor(ni2[:, samp2, :8])
    c4, s4 = _maxpool(_sc_gather(fm3, nip2), 256, w4, b4)
    verts2 = verts1[:, samp2, :]
    V2 = V1 // 8

    ni3 = _knn(verts2, NB + 1)[:, :, 1:]
    ni3t = _nmajor(ni3)
    v2pad = _pad16(verts2.reshape(-1, 3))
    gv3 = _sc_gather(v2pad, ni3t)

    gf4 = _sc_gather(s4, ni3t)
    c5, s5 = _agg(v2pad, gv3, d4, gf=gf4, center=c4, relu_out=True,
                  wl=w5, bl=b5)
    gf5 = _sc_gather(s5, ni3t)
    fm5 = _agg(v2pad, gv3, d5, gf=gf5, center=c5, relu_out=True)

    samp3 = jax.random.permutation(jax.random.key(303), V2)[:V2 // 8]
    nip3 = _nmajor(ni3[:, samp3, :8])
    c6, s6 = _maxpool(_gather_padded(fm5, nip3), 512, w6, b6, rows=bs * 4)
    verts3 = verts2[:, samp3, :]
    V3 = V2 // 8

    ni4 = _knn(verts3, 4)[:, :, 1:]
    ni4t = _nmajor(ni4)
    v3pad = _pad16(verts3.reshape(-1, 3))
    gv4 = _gather_padded(v3pad, ni4t)

    gf6 = _gather_padded(s6, ni4t)
    fg = _agg(v3pad, gv4, d6, gf=gf6, center=c6,
              reduce_rows=True, rows=bs * V3)
    return fg.reshape(bs, 1, 1024), vertices_anchor
